# baseline (speedup 1.0000x reference)
import functools

import jax
import jax.numpy as jnp
from jax import lax
from jax.experimental import pallas as pl
from jax.experimental.pallas import tpu as pltpu
from jax.experimental.pallas import tpu_sc as plsc

N = 100000
D = 128
B = 4096
MOM = 0.1
L = 16
NC = 2
NS = 16
NW = NC * NS
BP = B // NW
CROWS = 2000


def _rsqrt(t):
    i = plsc.bitcast(t, jnp.int32)
    i = jnp.int32(0x5F3759DF) - (i >> 1)
    y = plsc.bitcast(i, jnp.float32)
    for _ in range(3):
        y = y * (1.5 - 0.5 * t * y * y)
    return y


_SC_MESH = plsc.VectorSubcoreMesh(
    core_axis_name="c", subcore_axis_name="s", num_cores=NC, num_subcores=NS)
_SC_PARAMS = pltpu.CompilerParams(needs_layout_passes=False)


@functools.partial(
    pl.kernel,
    out_type=jax.ShapeDtypeStruct((B, D), jnp.float32),
    mesh=_SC_MESH,
    compiler_params=_SC_PARAMS,
    scratch_types=[
        pltpu.VMEM((BP,), jnp.int32),
        pltpu.VMEM((BP, D), jnp.float32),
        pltpu.VMEM((BP, D), jnp.float32),
        pltpu.SemaphoreType.DMA,
        pltpu.SemaphoreType.DMA,
    ],
)
def _sc_compute(inputs_hbm, targets_hbm, features_hbm, new_hbm,
                tgt_v, xbuf_v, obuf_v, gsem, xsem):
    wid = lax.axis_index("s") * NC + lax.axis_index("c")
    base = wid * BP

    pltpu.sync_copy(targets_hbm.at[pl.ds(base, BP)], tgt_v)
    g = pltpu.async_copy(features_hbm.at[tgt_v], obuf_v, gsem)
    x = pltpu.async_copy(inputs_hbm.at[pl.ds(base, BP)], xbuf_v, xsem)
    g.wait()
    x.wait()

    def row_body(r, _):
        acc = jnp.zeros((L,), jnp.float32)
        for f in range(D // L):
            old = obuf_v[r, pl.ds(f * L, L)]
            xv = xbuf_v[r, pl.ds(f * L, L)]
            nv = MOM * old + (1.0 - MOM) * xv
            obuf_v[r, pl.ds(f * L, L)] = nv
            acc = acc + nv * nv
        y = _rsqrt(jnp.broadcast_to(jnp.sum(acc), (L,)))
        for f in range(D // L):
            obuf_v[r, pl.ds(f * L, L)] = obuf_v[r, pl.ds(f * L, L)] * y
        return 0

    lax.fori_loop(0, BP, row_body, 0)
    pltpu.sync_copy(obuf_v, new_hbm.at[pl.ds(base, BP)])


@functools.partial(
    pl.kernel,
    out_type=(),
    mesh=_SC_MESH,
    compiler_params=_SC_PARAMS,
    scratch_types=[
        pltpu.VMEM((BP,), jnp.int32),
        pltpu.VMEM((BP, D), jnp.float32),
        pltpu.SemaphoreType.DMA,
    ],
)
def _sc_scatter(new_hbm, targets_hbm, bank_hbm, tgt_v, nbuf_v, gsem):
    wid = lax.axis_index("s") * NC + lax.axis_index("c")
    base = wid * BP
    pltpu.sync_copy(targets_hbm.at[pl.ds(base, BP)], tgt_v)
    pltpu.sync_copy(new_hbm.at[pl.ds(base, BP)], nbuf_v)
    pltpu.async_copy(nbuf_v, bank_hbm.at[tgt_v], gsem).wait()


def _copy_body(in_ref, out_ref):
    out_ref[...] = in_ref[...]


_tc_copy = pl.pallas_call(
    _copy_body,
    out_shape=jax.ShapeDtypeStruct((N, D), jnp.float32),
    grid=(N // CROWS,),
    in_specs=[pl.BlockSpec((CROWS, D), lambda i: (i, 0))],
    out_specs=pl.BlockSpec((CROWS, D), lambda i: (i, 0)),
)


def kernel(inputs, targets, features):
    tgt = targets.astype(jnp.int32)
    new = _sc_compute(inputs, tgt, features)
    bank = jax.new_ref(_tc_copy(features))
    _sc_scatter(new, tgt, bank)
    return bank[...]


# speedup vs baseline: 1.1323x; 1.1323x over previous
import functools

import jax
import jax.numpy as jnp
from jax import lax
from jax.experimental import pallas as pl
from jax.experimental.pallas import tpu as pltpu
from jax.experimental.pallas import tpu_sc as plsc

N = 100000
D = 128
B = 4096
MOM = 0.1
L = 16
NC = 2
NS = 16
NW = NC * NS
BP = B // NW


def _rsqrt(t):
    i = plsc.bitcast(t, jnp.int32)
    i = jnp.int32(0x5F3759DF) - (i >> 1)
    y = plsc.bitcast(i, jnp.float32)
    for _ in range(3):
        y = y * (1.5 - 0.5 * t * y * y)
    return y


_SC_MESH = plsc.VectorSubcoreMesh(
    core_axis_name="c", subcore_axis_name="s", num_cores=NC, num_subcores=NS)
_SC_PARAMS = pltpu.CompilerParams(needs_layout_passes=False)


@functools.partial(
    pl.kernel,
    out_type=jax.ShapeDtypeStruct((B, D), jnp.float32),
    mesh=_SC_MESH,
    compiler_params=_SC_PARAMS,
    scratch_types=[
        pltpu.VMEM((BP,), jnp.int32),
        pltpu.VMEM((BP, D), jnp.float32),
        pltpu.VMEM((BP, D), jnp.float32),
        pltpu.SemaphoreType.DMA,
        pltpu.SemaphoreType.DMA,
    ],
)
def _sc_compute(inputs_hbm, targets_hbm, features_hbm, new_hbm,
                tgt_v, xbuf_v, obuf_v, gsem, xsem):
    wid = lax.axis_index("s") * NC + lax.axis_index("c")
    base = wid * BP

    pltpu.sync_copy(targets_hbm.at[pl.ds(base, BP)], tgt_v)
    g = pltpu.async_copy(features_hbm.at[tgt_v], obuf_v, gsem)
    x = pltpu.async_copy(inputs_hbm.at[pl.ds(base, BP)], xbuf_v, xsem)
    g.wait()
    x.wait()

    def row_body(r, _):
        acc = jnp.zeros((L,), jnp.float32)
        for f in range(D // L):
            old = obuf_v[r, pl.ds(f * L, L)]
            xv = xbuf_v[r, pl.ds(f * L, L)]
            nv = MOM * old + (1.0 - MOM) * xv
            obuf_v[r, pl.ds(f * L, L)] = nv
            acc = acc + nv * nv
        y = _rsqrt(jnp.broadcast_to(jnp.sum(acc), (L,)))
        for f in range(D // L):
            obuf_v[r, pl.ds(f * L, L)] = obuf_v[r, pl.ds(f * L, L)] * y
        return 0

    lax.fori_loop(0, BP, row_body, 0)
    pltpu.sync_copy(obuf_v, new_hbm.at[pl.ds(base, BP)])


@functools.partial(
    pl.kernel,
    out_type=(),
    mesh=_SC_MESH,
    compiler_params=_SC_PARAMS,
    scratch_types=[
        pltpu.VMEM((BP,), jnp.int32),
        pltpu.VMEM((BP, D), jnp.float32),
        pltpu.SemaphoreType.DMA,
    ],
)
def _sc_scatter(new_hbm, targets_hbm, bank_hbm, tgt_v, nbuf_v, gsem):
    wid = lax.axis_index("s") * NC + lax.axis_index("c")
    base = wid * BP
    pltpu.sync_copy(targets_hbm.at[pl.ds(base, BP)], tgt_v)
    pltpu.sync_copy(new_hbm.at[pl.ds(base, BP)], nbuf_v)
    pltpu.async_copy(nbuf_v, bank_hbm.at[tgt_v], gsem).wait()


def kernel(inputs, targets, features):
    tgt = targets.astype(jnp.int32)
    new = _sc_compute(inputs, tgt, features)
    bank = jax.new_ref(features)
    _sc_scatter(new, tgt, bank)
    return bank[...]


# speedup vs baseline: 1.3105x; 1.1573x over previous
import functools

import jax
import jax.numpy as jnp
from jax import lax
from jax.experimental import pallas as pl
from jax.experimental.pallas import tpu as pltpu
from jax.experimental.pallas import tpu_sc as plsc

N = 100000
D = 128
B = 4096
MOM = 0.1
L = 16
NC = 2
NS = 16
NW = NC * NS
BP = B // NW


def _rsqrt(t):
    i = plsc.bitcast(t, jnp.int32)
    i = jnp.int32(0x5F3759DF) - (i >> 1)
    y = plsc.bitcast(i, jnp.float32)
    for _ in range(3):
        y = y * (1.5 - 0.5 * t * y * y)
    return y


@functools.partial(
    pl.kernel,
    out_type=(),
    mesh=plsc.VectorSubcoreMesh(
        core_axis_name="c", subcore_axis_name="s",
        num_cores=NC, num_subcores=NS),
    compiler_params=pltpu.CompilerParams(needs_layout_passes=False),
    scratch_types=[
        pltpu.VMEM((BP,), jnp.int32),
        pltpu.VMEM((BP, D), jnp.float32),
        pltpu.VMEM((BP, D), jnp.float32),
        pltpu.SemaphoreType.DMA,
        pltpu.SemaphoreType.DMA,
    ],
)
def _mb_update(inputs_hbm, targets_hbm, features_hbm, bank_hbm,
               tgt_v, xbuf_v, obuf_v, gsem, xsem):
    wid = lax.axis_index("s") * NC + lax.axis_index("c")
    base = wid * BP

    pltpu.sync_copy(targets_hbm.at[pl.ds(base, BP)], tgt_v)
    g = pltpu.async_copy(features_hbm.at[tgt_v], obuf_v, gsem)
    x = pltpu.async_copy(inputs_hbm.at[pl.ds(base, BP)], xbuf_v, xsem)
    g.wait()
    x.wait()

    UNROLL = 2

    def row_body(r2, _):
        for k in range(UNROLL):
            r = r2 * UNROLL + k
            acc = jnp.zeros((L,), jnp.float32)
            nvs = []
            for f in range(D // L):
                old = obuf_v[r, pl.ds(f * L, L)]
                xv = xbuf_v[r, pl.ds(f * L, L)]
                nv = MOM * old + (1.0 - MOM) * xv
                nvs.append(nv)
                acc = acc + nv * nv
            y = _rsqrt(jnp.broadcast_to(jnp.sum(acc), (L,)))
            for f in range(D // L):
                obuf_v[r, pl.ds(f * L, L)] = nvs[f] * y
        return 0

    lax.fori_loop(0, BP // UNROLL, row_body, 0)
    pltpu.async_copy(obuf_v, bank_hbm.at[tgt_v], gsem).wait()


def kernel(inputs, targets, features):
    bank = jax.new_ref(features)
    _mb_update(inputs, targets.astype(jnp.int32), features, bank)
    return bank[...]


# speedup vs baseline: 1.3107x; 1.0001x over previous
import functools

import jax
import jax.numpy as jnp
from jax import lax
from jax.experimental import pallas as pl
from jax.experimental.pallas import tpu as pltpu
from jax.experimental.pallas import tpu_sc as plsc

N = 100000
D = 128
B = 4096
MOM = 0.1
L = 16
NC = 2
NS = 16
NW = NC * NS
BP = B // NW


def _rsqrt(t):
    i = plsc.bitcast(t, jnp.int32)
    i = jnp.int32(0x5F3759DF) - (i >> 1)
    y = plsc.bitcast(i, jnp.float32)
    for _ in range(3):
        y = y * (1.5 - 0.5 * t * y * y)
    return y


@functools.partial(
    pl.kernel,
    out_type=(),
    mesh=plsc.VectorSubcoreMesh(
        core_axis_name="c", subcore_axis_name="s",
        num_cores=NC, num_subcores=NS),
    compiler_params=pltpu.CompilerParams(needs_layout_passes=False),
    scratch_types=[
        pltpu.VMEM((BP,), jnp.int32),
        pltpu.VMEM((BP, D), jnp.float32),
        pltpu.VMEM((BP, D), jnp.float32),
        pltpu.SemaphoreType.DMA,
        pltpu.SemaphoreType.DMA,
        pltpu.SemaphoreType.DMA,
    ],
)
def _mb_update(inputs_hbm, targets_hbm, features_hbm, bank_hbm,
               tgt_v, xbuf_v, obuf_v, gsem, xsem, hsem):
    wid = lax.axis_index("s") * NC + lax.axis_index("c")
    base = wid * BP

    H = BP // 2
    x = pltpu.async_copy(inputs_hbm.at[pl.ds(base, BP)], xbuf_v, xsem)
    pltpu.sync_copy(targets_hbm.at[pl.ds(base, BP)], tgt_v)
    g0 = pltpu.async_copy(features_hbm.at[tgt_v.at[pl.ds(0, H)]],
                          obuf_v.at[pl.ds(0, H)], gsem)
    g1 = pltpu.async_copy(features_hbm.at[tgt_v.at[pl.ds(H, H)]],
                          obuf_v.at[pl.ds(H, H)], hsem)
    g0.wait()
    x.wait()

    UNROLL = 2

    def make_body(r0):
        def row_body(r2, _):
            for k in range(UNROLL):
                r = r0 + r2 * UNROLL + k
                acc = jnp.zeros((L,), jnp.float32)
                nvs = []
                for f in range(D // L):
                    old = obuf_v[r, pl.ds(f * L, L)]
                    xv = xbuf_v[r, pl.ds(f * L, L)]
                    nv = MOM * old + (1.0 - MOM) * xv
                    nvs.append(nv)
                    acc = acc + nv * nv
                y = _rsqrt(jnp.broadcast_to(jnp.sum(acc), (L,)))
                for f in range(D // L):
                    obuf_v[r, pl.ds(f * L, L)] = nvs[f] * y
            return 0
        return row_body

    lax.fori_loop(0, H // UNROLL, make_body(0), 0)
    g1.wait()
    lax.fori_loop(0, H // UNROLL, make_body(H), 0)
    pltpu.async_copy(obuf_v, bank_hbm.at[tgt_v], gsem).wait()


def kernel(inputs, targets, features):
    bank = jax.new_ref(features)
    _mb_update(inputs, targets.astype(jnp.int32), features, bank)
    return bank[...]


# speedup vs baseline: 1.3113x; 1.0005x over previous
import functools

import jax
import jax.numpy as jnp
from jax import lax
from jax.experimental import pallas as pl
from jax.experimental.pallas import tpu as pltpu
from jax.experimental.pallas import tpu_sc as plsc

N = 100000
D = 128
B = 4096
MOM = 0.1
L = 16
NC = 2
NS = 16
NW = NC * NS
BP = B // NW


def _rsqrt(t):
    i = plsc.bitcast(t, jnp.int32)
    i = jnp.int32(0x5F3759DF) - (i >> 1)
    y = plsc.bitcast(i, jnp.float32)
    for _ in range(3):
        y = y * (1.5 - 0.5 * t * y * y)
    return y


@functools.partial(
    pl.kernel,
    out_type=(),
    mesh=plsc.VectorSubcoreMesh(
        core_axis_name="c", subcore_axis_name="s",
        num_cores=NC, num_subcores=NS),
    compiler_params=pltpu.CompilerParams(needs_layout_passes=False),
    scratch_types=[
        pltpu.VMEM((BP,), jnp.int32),
        pltpu.VMEM((BP, D), jnp.float32),
        pltpu.VMEM((BP, D), jnp.float32),
        pltpu.SemaphoreType.DMA,
        pltpu.SemaphoreType.DMA,
        pltpu.SemaphoreType.DMA,
    ],
)
def _mb_update(inputs_hbm, targets_hbm, features_hbm, bank_hbm,
               tgt_v, xbuf_v, obuf_v, gsem, xsem, hsem):
    wid = lax.axis_index("s") * NC + lax.axis_index("c")
    base = wid * BP

    H = BP // 2
    x = pltpu.async_copy(inputs_hbm.at[pl.ds(base, BP)], xbuf_v, xsem)
    pltpu.sync_copy(targets_hbm.at[pl.ds(base, BP)], tgt_v)
    g0 = pltpu.async_copy(features_hbm.at[tgt_v.at[pl.ds(0, H)]],
                          obuf_v.at[pl.ds(0, H)], gsem)
    g1 = pltpu.async_copy(features_hbm.at[tgt_v.at[pl.ds(H, H)]],
                          obuf_v.at[pl.ds(H, H)], hsem)
    g0.wait()
    x.wait()

    UNROLL = 4

    def make_body(r0):
        def row_body(r2, _):
            for k in range(UNROLL):
                r = r0 + r2 * UNROLL + k
                acc = jnp.zeros((L,), jnp.float32)
                nvs = []
                for f in range(D // L):
                    old = obuf_v[r, pl.ds(f * L, L)]
                    xv = xbuf_v[r, pl.ds(f * L, L)]
                    nv = MOM * old + (1.0 - MOM) * xv
                    nvs.append(nv)
                    acc = acc + nv * nv
                y = _rsqrt(jnp.broadcast_to(jnp.sum(acc), (L,)))
                for f in range(D // L):
                    obuf_v[r, pl.ds(f * L, L)] = nvs[f] * y
            return 0
        return row_body

    lax.fori_loop(0, H // UNROLL, make_body(0), 0)
    g1.wait()
    lax.fori_loop(0, H // UNROLL, make_body(H), 0)
    pltpu.async_copy(obuf_v, bank_hbm.at[tgt_v], gsem).wait()


def kernel(inputs, targets, features):
    bank = jax.new_ref(features)
    _mb_update(inputs, targets.astype(jnp.int32), features, bank)
    return bank[...]
